# baseline (speedup 1.0000x reference)
import jax
import jax.numpy as jnp
from jax import lax
from jax.experimental import pallas as pl
from jax.experimental.pallas import tpu as pltpu
from jax.experimental.pallas import tpu_sc as plsc

D = 128
NC = 2
NS = 16
LANES = 16
NW = NC * NS
NPAD = 10112
STRIPE = NPAD // NS
DENP = 10112
EPW = 10112
CHUNK = 64
NCHUNKS = EPW // CHUNK


def _mm_body(x_ref, wt_ref, a8_ref, z_ref, al8_ref):
    z = jnp.dot(x_ref[...], wt_ref[...], preferred_element_type=jnp.float32)
    z_ref[...] = z
    al8_ref[...] = jnp.dot(z, a8_ref[...], preferred_element_type=jnp.float32)


def _sc_body(z_hbm, al1_hbm, al2_hbm, edges_hbm, acc_hbm, den_hbm,
             al1_v, al2_v, idx_v, rows_v, w_v, den_v,
             gsem0, gsem1, isem0, isem1, isem2, ssem0, ssem1, acc_s):
    cid = lax.axis_index("c")
    sid = lax.axis_index("s")
    wid = cid * NS + sid
    npairs = edges_hbm.shape[1]
    chunk = edges_hbm.shape[4]
    nchunks = 2 * npairs
    gsems = (gsem0, gsem1)
    isems = (isem0, isem1, isem2)
    ssems = (ssem0, ssem1)

    zeros16 = jnp.zeros((LANES,), jnp.float32)

    pltpu.async_copy(al1_hbm, al1_v, isem0)
    pltpu.async_copy(al2_hbm, al2_v, isem2)
    pltpu.async_copy(edges_hbm.at[wid, 0], idx_v.at[0], gsem1)
    pltpu.async_copy(edges_hbm.at[wid, 1], idx_v.at[1], isem1)

    def zero_rows(c, _):
        for k in range(D // LANES):
            rows_v[0, c, pl.ds(k * LANES, LANES)] = zeros16
        return 0
    lax.fori_loop(0, chunk, zero_rows, 0)

    def zero_den(c, _):
        den_v[pl.ds(c * LANES, LANES)] = zeros16
        return 0
    lax.fori_loop(0, DENP // LANES, zero_den, 0)

    base = sid * STRIPE
    off = 0
    while off < STRIPE:
        cnt = min(chunk, STRIPE - off)
        pltpu.sync_copy(rows_v.at[0, pl.ds(0, cnt)],
                        acc_s.at[pl.ds(base + off, cnt)])
        off += cnt

    pltpu.make_async_copy(edges_hbm.at[wid, 0], idx_v.at[0], gsem1).wait()
    pltpu.async_copy(z_hbm.at[idx_v.at[0, 0, 0]], rows_v.at[0], gsem0)
    pltpu.make_async_copy(al1_hbm, al1_v, isem0).wait()
    pltpu.make_async_copy(al2_hbm, al2_v, isem2).wait()

    plsc.subcore_barrier()

    def pipe_step(j, b, ps, nps, first, last):
        nb = 1 - b
        if not last:
            def drain_nb():
                pltpu.make_async_copy(rows_v.at[nb],
                                      acc_s.at[idx_v.at[0, 0, 1]],
                                      ssems[nb]).wait()
            if first:
                pass
            else:
                drain_nb()
            pltpu.async_copy(z_hbm.at[idx_v.at[nps, (b + 1) % 2, 0]],
                             rows_v.at[nb], gsems[nb])

        pltpu.make_async_copy(z_hbm.at[idx_v.at[ps, b, 0]], rows_v.at[b],
                              gsems[b]).wait()

        for i in range(chunk // LANES):
            sl = pl.ds(i * LANES, LANES)
            sidx = idx_v[ps, b, 0, sl]
            didx = idx_v[ps, b, 1, sl]
            a1 = plsc.load_gather(al1_v, [sidx])
            a2 = plsc.load_gather(al2_v, [didx])
            e = a1 + a2
            e = jnp.where(e >= 0.0, e, 0.01 * e)
            w = jnp.exp(e)
            w_v[sl] = w
            plsc.addupdate_scatter(den_v, [didx], w)

        def scale_rows(c4, _):
            c0 = c4 * 4
            wcs = [plsc.load_gather(
                w_v, [jnp.full((LANES,), c0 + r, jnp.int32)])
                for r in range(4)]
            for k in range(D // LANES):
                sl = pl.ds(k * LANES, LANES)
                for r in range(4):
                    rows_v[b, c0 + r, sl] = rows_v[b, c0 + r, sl] * wcs[r]
            return 0
        lax.fori_loop(0, chunk // 4, scale_rows, 0)

        pltpu.async_copy(rows_v.at[b], acc_s.at[idx_v.at[ps, b, 1]],
                         ssems[b], add=True)

    def pair_step(p, ps, first, last):
        ps1 = (ps + 1) % 3
        ps2 = (ps + 2) % 3
        pipe_step(2 * p, 0, ps, ps, first, False)
        if not last:
            pltpu.make_async_copy(edges_hbm.at[wid, p + 1],
                                  idx_v.at[ps1], isems[ps1]).wait()
            pipe_step(2 * p + 1, 1, ps, ps1, False, False)
            @pl.when(jnp.asarray(p) + 2 < npairs)
            def _():
                pltpu.async_copy(edges_hbm.at[wid, p + 2], idx_v.at[ps2],
                                 isems[ps2])
        else:
            pipe_step(2 * p + 1, 1, ps, ps, False, True)

    def triple_body(q, _):
        p = 3 * q + 1
        pair_step(p, 1, False, False)
        pair_step(p + 1, 2, False, False)
        pair_step(p + 2, 0, False, False)
        return 0

    pair_step(0, 0, True, False)
    triples = (npairs - 2) // 3
    lax.fori_loop(0, triples, triple_body, 0)
    for p in range(1 + 3 * triples, npairs):
        pair_step(p, p % 3, False, p == npairs - 1)
    for b in (0, 1):
        pltpu.make_async_copy(rows_v.at[b], acc_s.at[idx_v.at[0, 0, 1]],
                              ssems[b]).wait()

    pltpu.sync_copy(den_v, den_hbm.at[wid])

    plsc.subcore_barrier()

    pltpu.sync_copy(acc_s.at[pl.ds(base, STRIPE)],
                    acc_hbm.at[cid, pl.ds(base, STRIPE)])


def _combine_body(a0_ref, a1_ref, dp_ref, o_ref):
    s = a0_ref[0] + a1_ref[0]
    den = jnp.sum(dp_ref[...], axis=0)
    o_ref[...] = s / den


def kernel(x, edge_index, W, A):
    n, d_in = x.shape
    d_out = W.shape[0]
    e_total = edge_index.shape[1]
    nchunks = NCHUNKS
    chunk = CHUNK

    wt = W.T
    a8 = jnp.pad(jnp.stack([A[0, :d_out], A[0, d_out:]], axis=1),
                 ((0, 0), (0, 6)))

    z, al8 = pl.pallas_call(
        _mm_body,
        out_shape=[
            jax.ShapeDtypeStruct((n, d_out), jnp.float32),
            jax.ShapeDtypeStruct((n, 8), jnp.float32),
        ],
    )(x, wt, a8)
    al1 = al8[:, 0]
    al2 = al8[:, 1]

    npad_e = NW * EPW - e_total
    al1p = jnp.concatenate([al1, jnp.zeros((DENP - n,), jnp.float32)])
    al2p = jnp.concatenate([al2, jnp.zeros((DENP - n,), jnp.float32)])
    src = jnp.concatenate(
        [edge_index[0].astype(jnp.int32),
         jnp.zeros((npad_e,), jnp.int32)]).reshape(NW, nchunks, 1, chunk)
    dst = jnp.concatenate(
        [edge_index[1].astype(jnp.int32),
         jnp.full((npad_e,), DENP - 1, jnp.int32)]).reshape(
             NW, nchunks, 1, chunk)
    edges = jnp.concatenate([src, dst], axis=2).reshape(
        NW, nchunks // 2, 2, 2, chunk)

    sc = pl.kernel(
        _sc_body,
        out_type=[
            jax.ShapeDtypeStruct((NC, NPAD, D), jnp.float32),
            jax.ShapeDtypeStruct((NW, DENP), jnp.float32),
        ],
        mesh=plsc.VectorSubcoreMesh(
            core_axis_name="c", subcore_axis_name="s",
            num_cores=NC, num_subcores=NS),
        compiler_params=pltpu.CompilerParams(needs_layout_passes=False),
        scratch_types=[
            pltpu.VMEM((DENP,), jnp.float32),
            pltpu.VMEM((DENP,), jnp.float32),
            pltpu.VMEM((3, 2, 2, chunk), jnp.int32),
            pltpu.VMEM((2, chunk, D), jnp.float32),
            pltpu.VMEM((chunk,), jnp.float32),
            pltpu.VMEM((DENP,), jnp.float32),
            pltpu.SemaphoreType.DMA,
            pltpu.SemaphoreType.DMA,
            pltpu.SemaphoreType.DMA,
            pltpu.SemaphoreType.DMA,
            pltpu.SemaphoreType.DMA,
            pltpu.SemaphoreType.DMA,
            pltpu.SemaphoreType.DMA,
            pltpu.VMEM_SHARED((NPAD, D), jnp.float32),
        ],
    )
    acc, den_part = sc(z, al1p, al2p, edges)
    den3 = den_part.reshape(NW, DENP, 1)

    blk = 1000
    h = pl.pallas_call(
        _combine_body,
        grid=(n // blk,),
        in_specs=[
            pl.BlockSpec((1, blk, D), lambda i: (0, i, 0)),
            pl.BlockSpec((1, blk, D), lambda i: (1, i, 0)),
            pl.BlockSpec((NW, blk, 1), lambda i: (0, i, 0)),
        ],
        out_specs=pl.BlockSpec((blk, d_out), lambda i: (i, 0)),
        out_shape=jax.ShapeDtypeStruct((n, d_out), jnp.float32),
    )(acc, acc, den3)
    return h
